# baseline (speedup 1.0000x reference)
import functools

import jax
import jax.numpy as jnp
from jax import lax
from jax.experimental import pallas as pl
from jax.experimental.pallas import tpu as pltpu
from jax.experimental.pallas import tpu_sc as plsc

MAX_NORM = 1.0
LANES = 16


def _vrsqrt(s):
    i = lax.bitcast_convert_type(s, jnp.int32)
    y = lax.bitcast_convert_type(
        jnp.int32(0x5F3759DF) - lax.shift_right_logical(i, 1), jnp.float32)
    half = 0.5 * s
    for _ in range(3):
        y = y * (1.5 - half * y * y)
    return y


def _sc_pool(idx_flat, table_pairs, B, L):
    VP, DP = table_pairs.shape
    D = 64
    n_chunk = D // LANES
    info = plsc.get_sparse_core_info()
    nc, ns = info.num_cores, info.num_subcores
    nw = nc * ns
    b_per_w = B // nw
    r_per_w = b_per_w * L
    nchunks = 4
    rows_per_chunk = r_per_w // nchunks
    b_per_chunk = b_per_w // nchunks
    mesh = plsc.VectorSubcoreMesh(core_axis_name="c", subcore_axis_name="s")

    @functools.partial(
        pl.kernel,
        mesh=mesh,
        compiler_params=pltpu.CompilerParams(
            use_tc_tiling_on_sc=True, needs_layout_passes=False),
        out_type=jax.ShapeDtypeStruct((B, D), jnp.float32),
        scratch_types=[
            pltpu.VMEM((r_per_w,), jnp.int32),
            pltpu.VMEM((r_per_w,), jnp.int32),
            pltpu.VMEM((2, rows_per_chunk, DP), jnp.float32),
            pltpu.VMEM((b_per_w, D), jnp.float32),
            pltpu.SemaphoreType.DMA,
            pltpu.SemaphoreType.DMA,
        ],
    )
    def k(table_hbm, idx_hbm, out_hbm, idx_v, idxp_v, bufs, x_v, sem0, sem1):
        wid = lax.axis_index("s") * nc + lax.axis_index("c")
        base = wid * r_per_w
        pltpu.sync_copy(idx_hbm.at[pl.ds(base, r_per_w)], idx_v)

        def halve_body(i, _):
            sl = pl.ds(i * LANES, LANES)
            idxp_v[sl] = lax.shift_right_logical(idx_v[sl], 1)
            return 0

        lax.fori_loop(0, r_per_w // LANES, halve_body, 0)
        sems = [sem0, sem1]

        def start(c):
            return pltpu.async_copy(
                table_hbm.at[idxp_v.at[pl.ds(c * rows_per_chunk,
                                             rows_per_chunk)]],
                bufs.at[c % 2], sems[c % 2])

        inv_l = jnp.float32(1.0 / L)
        lane = jnp.arange(LANES, dtype=jnp.int32)
        one = jnp.int32(1)
        handles = [start(0), start(1)]
        for c in range(nchunks):
            handles[c % 2].wait()
            buf = bufs.at[c % 2]

            def batch_body(bi, _, buf=buf, c=c):
                def row_body(li, accs):
                    r = bi * L + li
                    rg = c * rows_per_chunk + r
                    idxr = plsc.load_gather(
                        idx_v, [jnp.full((LANES,), rg, jnp.int32)])
                    colbase = lax.shift_left(idxr & one, 6) + lane
                    rows16 = jnp.full((LANES,), r, jnp.int32)
                    chunks = [plsc.load_gather(
                                  buf, [rows16, colbase + (cc * LANES)])
                              for cc in range(n_chunk)]
                    sq = chunks[0] * chunks[0]
                    for cc in range(1, n_chunk):
                        sq = sq + chunks[cc] * chunks[cc]
                    s = lax.reduce_sum_p.bind(sq, axes=(0,))
                    s_vec = jnp.full((LANES,), s, dtype=jnp.float32)
                    scale = jnp.minimum(
                        jnp.float32(MAX_NORM),
                        _vrsqrt(jnp.maximum(s_vec, jnp.float32(1e-14))))
                    return tuple(a + chunks[cc] * scale
                                 for cc, a in enumerate(accs))

                accs = lax.fori_loop(
                    0, L, row_body,
                    tuple(jnp.zeros((LANES,), jnp.float32)
                          for _ in range(n_chunk)))
                bg = c * b_per_chunk + bi
                for cc in range(n_chunk):
                    x_v[bg, pl.ds(cc * LANES, LANES)] = accs[cc] * inv_l
                return 0

            lax.fori_loop(0, b_per_chunk, batch_body, 0)
            if c + 2 < nchunks:
                handles[c % 2] = start(c + 2)
        pltpu.sync_copy(x_v, out_hbm.at[pl.ds(wid * b_per_w, b_per_w)])

    return k(table_pairs, idx_flat)


def _project(x, Wt, b2):
    B, D = x.shape
    V = Wt.shape[1]
    vt = 2048
    nv = pl.cdiv(V, vt)

    def body(w_ref, x_ref, b_ref, o_ref):
        acc = lax.dot_general(
            w_ref[...], x_ref[...], (((0,), (1,)), ((), ())),
            preferred_element_type=jnp.float32,
        )
        ones = jnp.ones((B, 1), jnp.float32)
        bias = lax.dot_general(
            b_ref[...], ones, (((0,), (1,)), ((), ())),
            preferred_element_type=jnp.float32,
        )
        o_ref[...] = acc + bias

    return pl.pallas_call(
        body,
        grid=(nv,),
        in_specs=[
            pl.BlockSpec((D, vt), lambda j: (0, j)),
            pl.BlockSpec((B, D), lambda j: (0, 0)),
            pl.BlockSpec((1, vt), lambda j: (0, j)),
        ],
        out_specs=pl.BlockSpec((vt, B), lambda j: (j, 0)),
        out_shape=jax.ShapeDtypeStruct((V, B), jnp.float32),
    )(Wt, x, b2)


def kernel(inputs_, emb_table, W, b):
    B, L = inputs_.shape
    V, D = emb_table.shape
    idx = inputs_.reshape(B * L).astype(jnp.int32)
    table_pairs = emb_table.reshape(V // 2, 2 * D)
    x = _sc_pool(idx, table_pairs, B, L)
    return _project(x, W.T, b.reshape(1, V)).T


# speedup vs baseline: 1.0106x; 1.0106x over previous
import functools

import jax
import jax.numpy as jnp
from jax import lax
from jax.experimental import pallas as pl
from jax.experimental.pallas import tpu as pltpu
from jax.experimental.pallas import tpu_sc as plsc

MAX_NORM = 1.0
LANES = 16


def _vrsqrt(s):
    i = lax.bitcast_convert_type(s, jnp.int32)
    y = lax.bitcast_convert_type(
        jnp.int32(0x5F3759DF) - lax.shift_right_logical(i, 1), jnp.float32)
    half = 0.5 * s
    for _ in range(3):
        y = y * (1.5 - half * y * y)
    return y


def _sc_pool(idx_flat, table_pairs, B, L):
    VP, DP = table_pairs.shape
    D = 64
    n_chunk = D // LANES
    info = plsc.get_sparse_core_info()
    nc, ns = info.num_cores, info.num_subcores
    nw = nc * ns
    b_per_w = B // nw
    r_per_w = b_per_w * L
    nchunks = 4
    rows_per_chunk = r_per_w // nchunks
    b_per_chunk = b_per_w // nchunks
    mesh = plsc.VectorSubcoreMesh(core_axis_name="c", subcore_axis_name="s")

    @functools.partial(
        pl.kernel,
        mesh=mesh,
        compiler_params=pltpu.CompilerParams(
            use_tc_tiling_on_sc=True, needs_layout_passes=False),
        out_type=jax.ShapeDtypeStruct((B, D), jnp.float32),
        scratch_types=[
            pltpu.VMEM((r_per_w,), jnp.int32),
            pltpu.VMEM((r_per_w,), jnp.int32),
            pltpu.VMEM((2, rows_per_chunk, DP), jnp.float32),
            pltpu.VMEM((b_per_w, D), jnp.float32),
            pltpu.SemaphoreType.DMA,
            pltpu.SemaphoreType.DMA,
        ],
    )
    def k(table_hbm, idx_hbm, out_hbm, idx_v, idxp_v, bufs, x_v, sem0, sem1):
        wid = lax.axis_index("s") * nc + lax.axis_index("c")
        base = wid * r_per_w
        pltpu.sync_copy(idx_hbm.at[pl.ds(base, r_per_w)], idx_v)

        def halve_body(i, _):
            sl = pl.ds(i * LANES, LANES)
            idxp_v[sl] = lax.shift_right_logical(idx_v[sl], 1)
            return 0

        lax.fori_loop(0, r_per_w // LANES, halve_body, 0)
        sems = [sem0, sem1]

        def start(c):
            return pltpu.async_copy(
                table_hbm.at[idxp_v.at[pl.ds(c * rows_per_chunk,
                                             rows_per_chunk)]],
                bufs.at[c % 2], sems[c % 2])

        inv_l = jnp.float32(1.0 / L)
        lane = jnp.arange(LANES, dtype=jnp.int32)
        one = jnp.int32(1)
        handles = [start(0), start(1)]
        for c in range(nchunks):
            handles[c % 2].wait()
            buf = bufs.at[c % 2]

            def batch_body(bi, _, buf=buf, c=c):
                def row_body(li, accs):
                    r = bi * L + li
                    rg = c * rows_per_chunk + r
                    idxr = plsc.load_gather(
                        idx_v, [jnp.full((LANES,), rg, jnp.int32)])
                    colbase = lax.shift_left(idxr & one, 6) + lane
                    rows16 = jnp.full((LANES,), r, jnp.int32)
                    chunks = [plsc.load_gather(
                                  buf, [rows16, colbase + (cc * LANES)])
                              for cc in range(n_chunk)]
                    sq = chunks[0] * chunks[0]
                    for cc in range(1, n_chunk):
                        sq = sq + chunks[cc] * chunks[cc]
                    s = lax.reduce_sum_p.bind(sq, axes=(0,))
                    s_vec = jnp.full((LANES,), s, dtype=jnp.float32)
                    scale = jnp.minimum(
                        jnp.float32(MAX_NORM),
                        _vrsqrt(jnp.maximum(s_vec, jnp.float32(1e-14))))
                    return tuple(a + chunks[cc] * scale
                                 for cc, a in enumerate(accs))

                accs = lax.fori_loop(
                    0, L, row_body,
                    tuple(jnp.zeros((LANES,), jnp.float32)
                          for _ in range(n_chunk)))
                bg = c * b_per_chunk + bi
                for cc in range(n_chunk):
                    x_v[bg, pl.ds(cc * LANES, LANES)] = accs[cc] * inv_l
                return 0

            lax.fori_loop(0, b_per_chunk, batch_body, 0)
            if c + 2 < nchunks:
                handles[c % 2] = start(c + 2)
        pltpu.sync_copy(x_v, out_hbm.at[pl.ds(wid * b_per_w, b_per_w)])

    return k(table_pairs, idx_flat)


def _project(x, Wt, b2):
    B, D = x.shape
    V = Wt.shape[1]
    vt = 4096
    nv = pl.cdiv(V, vt)

    def body(w_ref, x_ref, b_ref, o_ref):
        acc = lax.dot_general(
            w_ref[...], x_ref[...], (((0,), (1,)), ((), ())),
            preferred_element_type=jnp.float32,
        )
        ones = jnp.ones((B, 1), jnp.float32)
        bias = lax.dot_general(
            b_ref[...], ones, (((0,), (1,)), ((), ())),
            preferred_element_type=jnp.float32,
        )
        o_ref[...] = acc + bias

    return pl.pallas_call(
        body,
        grid=(nv,),
        in_specs=[
            pl.BlockSpec((D, vt), lambda j: (0, j)),
            pl.BlockSpec((B, D), lambda j: (0, 0)),
            pl.BlockSpec((1, vt), lambda j: (0, j)),
        ],
        out_specs=pl.BlockSpec((vt, B), lambda j: (j, 0)),
        out_shape=jax.ShapeDtypeStruct((V, B), jnp.float32),
    )(Wt, x, b2)


def kernel(inputs_, emb_table, W, b):
    B, L = inputs_.shape
    V, D = emb_table.shape
    idx = inputs_.reshape(B * L).astype(jnp.int32)
    table_pairs = emb_table.reshape(V // 2, 2 * D)
    x = _sc_pool(idx, table_pairs, B, L)
    return _project(x, W.T, b.reshape(1, V)).T
